# baseline (speedup 1.0000x reference)
import functools

import jax
import jax.numpy as jnp
from jax import lax
from jax.experimental import pallas as pl
from jax.experimental.pallas import tpu as pltpu
from jax.experimental.pallas import tpu_sc as plsc

N = 10000
E = 320000
NFEAT = 128
NHID = 128
NCLASS = 16
NGRAPHS = 128

NPAD = 10240
BLK = 512
NBLK = NPAD // BLK

NC = 2
NS = 16
NW = NC * NS
CHUNK = 128
NCHUNK = 79
EPAD = NW * NCHUNK * CHUNK
DUMMY = NPAD - 1
ROWS_PER_SUB = NPAD // NS

@functools.cache
def _sc_mesh():
    return plsc.VectorSubcoreMesh(core_axis_name="c", subcore_axis_name="s",
                                  num_cores=NC, num_subcores=NS)


def _zero_vmem(ref, rows, cols):
    @pl.loop(0, rows)
    def _(r):
        @pl.loop(0, cols, step=16)
        def _(cc):
            ref[r, pl.ds(cc, 16)] = jnp.zeros((16,), jnp.float32)


def _unpack_chunk(packed_ref, j, s_stage, d_stage):
    @pl.loop(0, CHUNK, step=16)
    def _(kk):
        pv = packed_ref[j, pl.ds(kk, 16)]
        s_stage[pl.ds(kk, 16)] = lax.bitwise_and(pv, jnp.int32(0xFFFF))
        d_stage[pl.ds(kk, 16)] = lax.shift_right_logical(pv, jnp.int32(16))


def _sc_degree(ed):
    @functools.partial(
        pl.kernel,
        out_type=jax.ShapeDtypeStruct((NC, NPAD, 128), jnp.float32),
        mesh=_sc_mesh(),
        scratch_types=[
            pltpu.VMEM((NCHUNK, CHUNK), jnp.int32),
            pltpu.VMEM((CHUNK,), jnp.int32),
            pltpu.VMEM((CHUNK,), jnp.int32),
            pltpu.VMEM((CHUNK, 128), jnp.float32),
            pltpu.VMEM((8, 128), jnp.float32),
            pltpu.VMEM_SHARED((NPAD, 128), jnp.float32),
        ],
    )
    def deg_kernel(e_hbm, out_hbm, e_v, d_st, s_st, ones_v, z_v, hist_sh):
        cid = lax.axis_index("c")
        sid = lax.axis_index("s")
        wid = sid * NC + cid

        @pl.loop(0, CHUNK)
        def _(r):
            @pl.loop(0, 128, step=16)
            def _(cc):
                ones_v[r, pl.ds(cc, 16)] = jnp.ones((16,), jnp.float32)
        _zero_vmem(z_v, 8, 128)

        @pl.loop(0, ROWS_PER_SUB, step=8)
        def _(k):
            pltpu.sync_copy(z_v, hist_sh.at[pl.ds(sid * ROWS_PER_SUB + k, 8)])
        plsc.subcore_barrier()

        pltpu.sync_copy(e_hbm.at[wid], e_v)

        @pl.loop(0, NCHUNK)
        def _(j):
            _unpack_chunk(e_v, j, s_st, d_st)
            pltpu.sync_copy(ones_v, hist_sh.at[d_st], add=True)
        plsc.subcore_barrier()

        @pl.loop(0, ROWS_PER_SUB, step=128)
        def _(k):
            r = sid * ROWS_PER_SUB + k
            pltpu.sync_copy(hist_sh.at[pl.ds(r, 128)],
                            out_hbm.at[cid, pl.ds(r, 128)])

    return deg_kernel(ed)


def _sc_edge_agg(u, ed):
    @functools.partial(
        pl.kernel,
        out_type=jax.ShapeDtypeStruct((NC, NPAD, 128), jnp.float32),
        mesh=_sc_mesh(),
        scratch_types=[
            pltpu.VMEM((NCHUNK, CHUNK), jnp.int32),
            pltpu.VMEM((CHUNK,), jnp.int32),
            pltpu.VMEM((CHUNK,), jnp.int32),
            pltpu.VMEM((CHUNK,), jnp.int32),
            pltpu.VMEM((CHUNK,), jnp.int32),
            pltpu.VMEM((CHUNK, 128), jnp.float32),
            pltpu.VMEM((CHUNK, 128), jnp.float32),
            pltpu.VMEM((8, 128), jnp.float32),
            pltpu.VMEM_SHARED((NPAD, 128), jnp.float32),
            pltpu.SemaphoreType.DMA,
            pltpu.SemaphoreType.DMA,
        ],
    )
    def agg_kernel(u_hbm, e_hbm, out_hbm, e_v, s0, d0, s1, d1, buf0, buf1,
                   z_v, acc_sh, sem0, sem1):
        cid = lax.axis_index("c")
        sid = lax.axis_index("s")
        wid = sid * NC + cid

        _zero_vmem(z_v, 8, 128)

        @pl.loop(0, ROWS_PER_SUB, step=8)
        def _(k):
            pltpu.sync_copy(z_v, acc_sh.at[pl.ds(sid * ROWS_PER_SUB + k, 8)])
        plsc.subcore_barrier()

        pltpu.sync_copy(e_hbm.at[wid], e_v)

        _unpack_chunk(e_v, 0, s0, d0)
        pltpu.async_copy(u_hbm.at[s0], buf0, sem0)

        @pl.loop(0, NCHUNK, step=2)
        def _(j):
            @pl.when(j + 1 < NCHUNK)
            def _():
                _unpack_chunk(e_v, j + 1, s1, d1)
                pltpu.async_copy(u_hbm.at[s1], buf1, sem1)
            pltpu.make_async_copy(u_hbm.at[s0], buf0, sem0).wait()
            pltpu.sync_copy(buf0, acc_sh.at[d0], add=True)

            @pl.when(j + 2 < NCHUNK)
            def _():
                _unpack_chunk(e_v, j + 2, s0, d0)
                pltpu.async_copy(u_hbm.at[s0], buf0, sem0)

            @pl.when(j + 1 < NCHUNK)
            def _():
                pltpu.make_async_copy(u_hbm.at[s1], buf1, sem1).wait()
                pltpu.sync_copy(buf1, acc_sh.at[d1], add=True)
        plsc.subcore_barrier()

        @pl.loop(0, ROWS_PER_SUB, step=128)
        def _(k):
            r = sid * ROWS_PER_SUB + k
            pltpu.sync_copy(acc_sh.at[pl.ds(r, 128)],
                            out_hbm.at[cid, pl.ds(r, 128)])

    return agg_kernel(u, ed)


def _tc_first(xp, W0, degp):
    def body(dp_ref, x_ref, w_ref, u_ref, dinv_ref):
        deg = dp_ref[0, :, 0:1] + dp_ref[1, :, 0:1] + 1.0
        dinv = lax.rsqrt(deg)
        dinv_ref[...] = dinv
        u_ref[...] = dinv * jnp.dot(x_ref[...], w_ref[...],
                                    preferred_element_type=jnp.float32)

    return pl.pallas_call(
        body,
        grid=(NBLK,),
        in_specs=[
            pl.BlockSpec((NC, BLK, 128), lambda i: (0, i, 0)),
            pl.BlockSpec((BLK, NFEAT), lambda i: (i, 0)),
            pl.BlockSpec((NFEAT, NHID), lambda i: (0, 0)),
        ],
        out_specs=[
            pl.BlockSpec((BLK, NHID), lambda i: (i, 0)),
            pl.BlockSpec((BLK, 1), lambda i: (i, 0)),
        ],
        out_shape=[
            jax.ShapeDtypeStruct((NPAD, NHID), jnp.float32),
            jax.ShapeDtypeStruct((NPAD, 1), jnp.float32),
        ],
    )(degp, xp, W0)


def _tc_layer(p, u, dinv, b, Wnext):
    def body(p_ref, u_ref, dinv_ref, b_ref, w_ref, out_ref):
        dinv = dinv_ref[...]
        h = jnp.maximum(dinv * (p_ref[0] + p_ref[1] + u_ref[...]) + b_ref[...],
                        0.0)
        out_ref[...] = dinv * jnp.dot(h, w_ref[...],
                                      preferred_element_type=jnp.float32)

    return pl.pallas_call(
        body,
        grid=(NBLK,),
        in_specs=[
            pl.BlockSpec((NC, BLK, NHID), lambda i: (0, i, 0)),
            pl.BlockSpec((BLK, NHID), lambda i: (i, 0)),
            pl.BlockSpec((BLK, 1), lambda i: (i, 0)),
            pl.BlockSpec((1, NHID), lambda i: (0, 0)),
            pl.BlockSpec((NHID, NHID), lambda i: (0, 0)),
        ],
        out_specs=pl.BlockSpec((BLK, NHID), lambda i: (i, 0)),
        out_shape=jax.ShapeDtypeStruct((NPAD, NHID), jnp.float32),
    )(p, u, dinv, b, Wnext)


def _tc_head(p, u, dinv, b, batch3, lin_W, lin_b):
    def body(p_ref, u_ref, dinv_ref, b_ref, bat_ref, lw_ref, lb_ref,
             out_ref, sums, cnts):
        i = pl.program_id(0)

        @pl.when(i == 0)
        def _():
            sums[...] = jnp.zeros_like(sums)
            cnts[...] = jnp.zeros_like(cnts)

        dinv = dinv_ref[...]
        h = jnp.maximum(dinv * (p_ref[0] + p_ref[1] + u_ref[...]) + b_ref[...],
                        0.0)
        seg = bat_ref[0, 0].reshape(BLK, 1)
        onehot = (seg == lax.broadcasted_iota(jnp.int32, (BLK, NGRAPHS), 1)
                  ).astype(jnp.float32)
        dn = (((0,), (0,)), ((), ()))
        sums[...] += lax.dot_general(onehot, h, dn,
                                     preferred_element_type=jnp.float32)
        cnts[...] += lax.dot_general(onehot, jnp.ones((BLK, NHID), jnp.float32),
                                     dn, preferred_element_type=jnp.float32)

        @pl.when(i == NBLK - 1)
        def _():
            pooled = sums[...] / jnp.maximum(cnts[...], 1.0)
            logits = jnp.dot(pooled, lw_ref[...],
                             preferred_element_type=jnp.float32) + lb_ref[...]
            m = jnp.max(logits, axis=-1, keepdims=True)
            ls = logits - m
            out_ref[...] = ls - jnp.log(jnp.sum(jnp.exp(ls), axis=-1,
                                                keepdims=True))

    return pl.pallas_call(
        body,
        grid=(NBLK,),
        in_specs=[
            pl.BlockSpec((NC, BLK, NHID), lambda i: (0, i, 0)),
            pl.BlockSpec((BLK, NHID), lambda i: (i, 0)),
            pl.BlockSpec((BLK, 1), lambda i: (i, 0)),
            pl.BlockSpec((1, NHID), lambda i: (0, 0)),
            pl.BlockSpec((1, 1, BLK), lambda i: (i, 0, 0)),
            pl.BlockSpec((NHID, NCLASS), lambda i: (0, 0)),
            pl.BlockSpec((1, NCLASS), lambda i: (0, 0)),
        ],
        out_specs=pl.BlockSpec((NGRAPHS, NCLASS), lambda i: (0, 0)),
        out_shape=jax.ShapeDtypeStruct((NGRAPHS, NCLASS), jnp.float32),
        scratch_shapes=[
            pltpu.VMEM((NGRAPHS, NHID), jnp.float32),
            pltpu.VMEM((NGRAPHS, NHID), jnp.float32),
        ],
    )(p, u, dinv, b, batch3, lin_W, lin_b)


def kernel(x, edge_index, batch, W0, b0, W1, b1, W2, b2, lin_W, lin_b):
    ed = jnp.pad(lax.shift_left(edge_index[1], jnp.int32(16)) | edge_index[0],
                 (0, EPAD - E),
                 constant_values=(DUMMY << 16) | DUMMY
                 ).reshape(NW, NCHUNK, CHUNK)
    xp = jnp.pad(x, ((0, NPAD - N), (0, 0)))
    batch3 = jnp.pad(batch, (0, NPAD - N),
                     constant_values=NGRAPHS).reshape(NBLK, 1, BLK)

    degp = _sc_degree(ed)
    u0, dinv = _tc_first(xp, W0, degp)

    p0 = _sc_edge_agg(u0, ed)
    u1 = _tc_layer(p0, u0, dinv, b0.reshape(1, NHID), W1)

    p1 = _sc_edge_agg(u1, ed)
    u2 = _tc_layer(p1, u1, dinv, b1.reshape(1, NHID), W2)

    p2 = _sc_edge_agg(u2, ed)
    return _tc_head(p2, u2, dinv, b2.reshape(1, NHID), batch3,
                    lin_W, lin_b.reshape(1, NCLASS))


# speedup vs baseline: 1.9453x; 1.9453x over previous
import functools

import jax
import jax.numpy as jnp
from jax import lax
from jax.experimental import pallas as pl
from jax.experimental.pallas import tpu as pltpu
from jax.experimental.pallas import tpu_sc as plsc

N = 10000
E = 320000
NFEAT = 128
NHID = 128
NCLASS = 16
NGRAPHS = 128

NPAD = 10240
BLK = 512
NBLK = NPAD // BLK

NC = 2
NS = 16
NW = NC * NS
CHUNK = 128
NCHUNK = 79
EPAD = NW * NCHUNK * CHUNK
DUMMY = NPAD - 1
ROWS_PER_SUB = NPAD // NS

@functools.cache
def _sc_mesh():
    return plsc.VectorSubcoreMesh(core_axis_name="c", subcore_axis_name="s",
                                  num_cores=NC, num_subcores=NS)


def _zero_vmem(ref, rows, cols):
    @pl.loop(0, rows)
    def _(r):
        @pl.loop(0, cols, step=16)
        def _(cc):
            ref[r, pl.ds(cc, 16)] = jnp.zeros((16,), jnp.float32)


def _unpack_chunk(packed_ref, j, s_stage, d_stage):
    @pl.loop(0, CHUNK, step=16)
    def _(kk):
        pv = packed_ref[j, pl.ds(kk, 16)]
        s_stage[pl.ds(kk, 16)] = lax.bitwise_and(pv, jnp.int32(0xFFFF))
        d_stage[pl.ds(kk, 16)] = lax.shift_right_logical(pv, jnp.int32(16))


def _sc_degree(ed):
    @functools.partial(
        pl.kernel,
        out_type=jax.ShapeDtypeStruct((NC, NPAD, 128), jnp.float32),
        mesh=_sc_mesh(),
        scratch_types=[
            pltpu.VMEM((NCHUNK, CHUNK), jnp.int32),
            pltpu.VMEM((CHUNK,), jnp.int32),
            pltpu.VMEM((CHUNK,), jnp.int32),
            pltpu.VMEM((CHUNK, 128), jnp.float32),
            pltpu.VMEM((8, 128), jnp.float32),
            pltpu.VMEM_SHARED((NPAD, 128), jnp.float32),
        ],
    )
    def deg_kernel(e_hbm, out_hbm, e_v, d_st, s_st, ones_v, z_v, hist_sh):
        cid = lax.axis_index("c")
        sid = lax.axis_index("s")
        wid = sid * NC + cid

        @pl.loop(0, CHUNK)
        def _(r):
            @pl.loop(0, 128, step=16)
            def _(cc):
                ones_v[r, pl.ds(cc, 16)] = jnp.ones((16,), jnp.float32)
        _zero_vmem(z_v, 8, 128)

        @pl.loop(0, ROWS_PER_SUB, step=8)
        def _(k):
            pltpu.sync_copy(z_v, hist_sh.at[pl.ds(sid * ROWS_PER_SUB + k, 8)])
        plsc.subcore_barrier()

        pltpu.sync_copy(e_hbm.at[wid], e_v)

        @pl.loop(0, NCHUNK)
        def _(j):
            _unpack_chunk(e_v, j, s_st, d_st)
            pltpu.sync_copy(ones_v, hist_sh.at[d_st], add=True)
        plsc.subcore_barrier()

        @pl.loop(0, ROWS_PER_SUB, step=128)
        def _(k):
            r = sid * ROWS_PER_SUB + k
            pltpu.sync_copy(hist_sh.at[pl.ds(r, 128)],
                            out_hbm.at[cid, pl.ds(r, 128)])

    return deg_kernel(ed)


def _sc_edge_agg(u, ed):
    @functools.partial(
        pl.kernel,
        out_type=jax.ShapeDtypeStruct((NC, NPAD, 128), jnp.float32),
        mesh=_sc_mesh(),
        scratch_types=[
            pltpu.VMEM((NCHUNK, CHUNK), jnp.int32),
            pltpu.VMEM((CHUNK,), jnp.int32),
            pltpu.VMEM((CHUNK,), jnp.int32),
            pltpu.VMEM((CHUNK,), jnp.int32),
            pltpu.VMEM((CHUNK,), jnp.int32),
            pltpu.VMEM((CHUNK, 128), jnp.float32),
            pltpu.VMEM((CHUNK, 128), jnp.float32),
            pltpu.VMEM((8, 128), jnp.float32),
            pltpu.VMEM_SHARED((NPAD, 128), jnp.float32),
            pltpu.SemaphoreType.DMA,
            pltpu.SemaphoreType.DMA,
        ],
    )
    def agg_kernel(u_hbm, e_hbm, out_hbm, e_v, s0, d0, s1, d1, buf0, buf1,
                   z_v, acc_sh, sem0, sem1):
        cid = lax.axis_index("c")
        sid = lax.axis_index("s")
        wid = sid * NC + cid

        _zero_vmem(z_v, 8, 128)

        @pl.loop(0, ROWS_PER_SUB, step=8)
        def _(k):
            pltpu.sync_copy(z_v, acc_sh.at[pl.ds(sid * ROWS_PER_SUB + k, 8)])
        plsc.subcore_barrier()

        pltpu.sync_copy(e_hbm.at[wid], e_v)

        _unpack_chunk(e_v, 0, s0, d0)
        pltpu.async_copy(u_hbm.at[s0], buf0, sem0)

        @pl.loop(0, NCHUNK, step=2)
        def _(j):
            @pl.when(j + 1 < NCHUNK)
            def _():
                _unpack_chunk(e_v, j + 1, s1, d1)
                pltpu.async_copy(u_hbm.at[s1], buf1, sem1)
            pltpu.make_async_copy(u_hbm.at[s0], buf0, sem0).wait()
            pltpu.sync_copy(buf0, acc_sh.at[d0], add=True)

            @pl.when(j + 2 < NCHUNK)
            def _():
                _unpack_chunk(e_v, j + 2, s0, d0)
                pltpu.async_copy(u_hbm.at[s0], buf0, sem0)

            @pl.when(j + 1 < NCHUNK)
            def _():
                pltpu.make_async_copy(u_hbm.at[s1], buf1, sem1).wait()
                pltpu.sync_copy(buf1, acc_sh.at[d1], add=True)
        plsc.subcore_barrier()

        @pl.loop(0, ROWS_PER_SUB, step=128)
        def _(k):
            r = sid * ROWS_PER_SUB + k
            pltpu.sync_copy(acc_sh.at[pl.ds(r, 128)],
                            out_hbm.at[cid, pl.ds(r, 128)])

    return agg_kernel(u, ed)


def _tc_first(xp, W0, degp):
    def body(dp_ref, x_ref, w_ref, u_ref, dinv_ref):
        deg = dp_ref[0, :, 0:1] + dp_ref[1, :, 0:1] + 1.0
        dinv = lax.rsqrt(deg)
        dinv_ref[...] = dinv
        u_ref[...] = dinv * jnp.dot(x_ref[...], w_ref[...],
                                    preferred_element_type=jnp.float32)

    return pl.pallas_call(
        body,
        grid=(NBLK,),
        in_specs=[
            pl.BlockSpec((NC, BLK, 128), lambda i: (0, i, 0)),
            pl.BlockSpec((BLK, NFEAT), lambda i: (i, 0)),
            pl.BlockSpec((NFEAT, NHID), lambda i: (0, 0)),
        ],
        out_specs=[
            pl.BlockSpec((BLK, NHID), lambda i: (i, 0)),
            pl.BlockSpec((BLK, 1), lambda i: (i, 0)),
        ],
        out_shape=[
            jax.ShapeDtypeStruct((NPAD, NHID), jnp.float32),
            jax.ShapeDtypeStruct((NPAD, 1), jnp.float32),
        ],
    )(degp, xp, W0)


def _tc_layer(p, u, dinv, b, Wnext):
    def body(p_ref, u_ref, dinv_ref, b_ref, w_ref, out_ref):
        dinv = dinv_ref[...]
        h = jnp.maximum(dinv * (p_ref[0] + p_ref[1] + u_ref[...]) + b_ref[...],
                        0.0)
        out_ref[...] = dinv * jnp.dot(h, w_ref[...],
                                      preferred_element_type=jnp.float32)

    return pl.pallas_call(
        body,
        grid=(NBLK,),
        in_specs=[
            pl.BlockSpec((NC, BLK, NHID), lambda i: (0, i, 0)),
            pl.BlockSpec((BLK, NHID), lambda i: (i, 0)),
            pl.BlockSpec((BLK, 1), lambda i: (i, 0)),
            pl.BlockSpec((1, NHID), lambda i: (0, 0)),
            pl.BlockSpec((NHID, NHID), lambda i: (0, 0)),
        ],
        out_specs=pl.BlockSpec((BLK, NHID), lambda i: (i, 0)),
        out_shape=jax.ShapeDtypeStruct((NPAD, NHID), jnp.float32),
    )(p, u, dinv, b, Wnext)


def _tc_head(p, u, dinv, b, batch3, lin_W, lin_b):
    def body(p_ref, u_ref, dinv_ref, b_ref, bat_ref, lw_ref, lb_ref,
             out_ref, sums, cnts):
        i = pl.program_id(0)

        @pl.when(i == 0)
        def _():
            sums[...] = jnp.zeros_like(sums)
            cnts[...] = jnp.zeros_like(cnts)

        dinv = dinv_ref[...]
        h = jnp.maximum(dinv * (p_ref[0] + p_ref[1] + u_ref[...]) + b_ref[...],
                        0.0)
        seg = bat_ref[0, 0].reshape(BLK, 1)
        onehot = (seg == lax.broadcasted_iota(jnp.int32, (BLK, NGRAPHS), 1)
                  ).astype(jnp.float32)
        dn = (((0,), (0,)), ((), ()))
        sums[...] += lax.dot_general(onehot, h, dn,
                                     preferred_element_type=jnp.float32)
        cnts[...] += lax.dot_general(onehot, jnp.ones((BLK, NHID), jnp.float32),
                                     dn, preferred_element_type=jnp.float32)

        @pl.when(i == NBLK - 1)
        def _():
            pooled = sums[...] / jnp.maximum(cnts[...], 1.0)
            logits = jnp.dot(pooled, lw_ref[...],
                             preferred_element_type=jnp.float32) + lb_ref[...]
            m = jnp.max(logits, axis=-1, keepdims=True)
            ls = logits - m
            out_ref[...] = ls - jnp.log(jnp.sum(jnp.exp(ls), axis=-1,
                                                keepdims=True))

    return pl.pallas_call(
        body,
        grid=(NBLK,),
        in_specs=[
            pl.BlockSpec((NC, BLK, NHID), lambda i: (0, i, 0)),
            pl.BlockSpec((BLK, NHID), lambda i: (i, 0)),
            pl.BlockSpec((BLK, 1), lambda i: (i, 0)),
            pl.BlockSpec((1, NHID), lambda i: (0, 0)),
            pl.BlockSpec((1, 1, BLK), lambda i: (i, 0, 0)),
            pl.BlockSpec((NHID, NCLASS), lambda i: (0, 0)),
            pl.BlockSpec((1, NCLASS), lambda i: (0, 0)),
        ],
        out_specs=pl.BlockSpec((NGRAPHS, NCLASS), lambda i: (0, 0)),
        out_shape=jax.ShapeDtypeStruct((NGRAPHS, NCLASS), jnp.float32),
        scratch_shapes=[
            pltpu.VMEM((NGRAPHS, NHID), jnp.float32),
            pltpu.VMEM((NGRAPHS, NHID), jnp.float32),
        ],
    )(p, u, dinv, b, batch3, lin_W, lin_b)


def kernel(x, edge_index, batch, W0, b0, W1, b1, W2, b2, lin_W, lin_b):
    dummy_ids = N + (jnp.arange(EPAD - E, dtype=jnp.int32) % (NPAD - N))
    ed = jnp.concatenate(
        [lax.shift_left(edge_index[1], jnp.int32(16)) | edge_index[0],
         lax.shift_left(dummy_ids, jnp.int32(16)) | dummy_ids]
    ).reshape(NW, NCHUNK, CHUNK)
    xp = jnp.pad(x, ((0, NPAD - N), (0, 0)))
    batch3 = jnp.pad(batch, (0, NPAD - N),
                     constant_values=NGRAPHS).reshape(NBLK, 1, BLK)

    degp = _sc_degree(ed)
    u0, dinv = _tc_first(xp, W0, degp)

    p0 = _sc_edge_agg(u0, ed)
    u1 = _tc_layer(p0, u0, dinv, b0.reshape(1, NHID), W1)

    p1 = _sc_edge_agg(u1, ed)
    u2 = _tc_layer(p1, u1, dinv, b1.reshape(1, NHID), W2)

    p2 = _sc_edge_agg(u2, ed)
    return _tc_head(p2, u2, dinv, b2.reshape(1, NHID), batch3,
                    lin_W, lin_b.reshape(1, NCLASS))
